# baseline (speedup 1.0000x reference)
import functools
import math

import jax
import jax.numpy as jnp
from jax import lax
from jax.experimental import pallas as pl
from jax.experimental.pallas import tpu as pltpu
from jax.experimental.pallas import tpu_sc as plsc

N = 1000000
D = 64
K = 16
LAMBDA = 3.0

BK = 8192
NGRID = 123
NTOT = NGRID * BK
G = 256
GPB = BK // G
NB = NTOT // G
NB_V = NB // 16

_NT = (((1,), (1,)), ((), ()))


def _aug(tr, i):
    t = tr[...]
    u = jnp.concatenate([t * t, t], axis=1)
    idx = jax.lax.broadcasted_iota(jnp.int32, (1, BK), 1) + i * BK
    return u, idx < N


def _store_grouped(vals, out_ref, nrows):
    for g in range(GPB):
        out_ref[:, g, :] = vals[:, g * G:(g + 1) * G]


def _k1_body(xur, tr, qr, bmr):
    i = pl.program_id(0)
    u, valid = _aug(tr, i)
    qq = lax.dot_general(xur[...], u, _NT,
                         preferred_element_type=jnp.float32,
                         precision=jax.lax.Precision.HIGHEST)
    q = jnp.where(valid, qq[0:1, :], jnp.float32(jnp.inf))
    _store_grouped(q, qr, 1)
    mins = [jnp.min(q[:, g * G:(g + 1) * G], axis=1, keepdims=True)
            for g in range(GPB)]
    bmr[...] = jnp.concatenate(mins, axis=1)[None]


def _k1(xu, train):
    return pl.pallas_call(
        _k1_body,
        grid=(NGRID,),
        in_specs=[
            pl.BlockSpec((8, 2 * D), lambda i: (0, 0)),
            pl.BlockSpec((BK, D), lambda i: (i, 0)),
        ],
        out_specs=[
            pl.BlockSpec((1, GPB, G), lambda i: (i, 0, 0)),
            pl.BlockSpec((1, 1, GPB), lambda i: (i, 0, 0)),
        ],
        out_shape=[
            jax.ShapeDtypeStruct((NGRID, GPB, G), jnp.float32),
            jax.ShapeDtypeStruct((NGRID, 1, GPB), jnp.float32),
        ],
        compiler_params=pltpu.CompilerParams(
            dimension_semantics=("arbitrary",)),
    )(xu, train)


def _merge16(bv, bi, v, iv):
    sv, si = plsc.sort_key_val(v, iv)
    rv = lax.rev(sv, (0,))
    ri = lax.rev(si, (0,))
    take = bv <= rv
    lo = jnp.minimum(bv, rv)
    li = jnp.where(take, bi, ri)
    nbv, nbi = plsc.sort_key_val(lo, li)
    return nbv, nbi, nbv[15]


def _scan_topk(nvecs, getv, geti, carry):
    def body(j, c):
        bv, bi, th = c
        v = getv(j)
        pred = jnp.any(v < th)

        def do(_):
            return _merge16(bv, bi, v, geti(j))

        return lax.cond(pred, do, lambda _: (bv, bi, th), 0)

    return lax.fori_loop(0, nvecs, body, carry)


def _init_carry():
    inf = jnp.float32(jnp.inf)
    return (jnp.full((16,), inf, jnp.float32),
            jnp.zeros((16,), jnp.int32),
            inf)


_SC_PARAMS = pltpu.CompilerParams(
    needs_layout_passes=False, use_tc_tiling_on_sc=False)
_SC_MESH = dict(core_axis_name="c", subcore_axis_name="s")


def _k2_body(q2_hbm, qbm_hbm, train_hbm, p_out,
             bmv, gidv, cand, biv, rows, sem):
    c = lax.axis_index("c")
    s = lax.axis_index("s")
    wid = s * 2 + c

    @pl.when(wid == 0)
    def _():
        pltpu.sync_copy(qbm_hbm, bmv)
        iota16 = lax.iota(jnp.int32, 16)
        _, bi, _ = _scan_topk(
            NB_V,
            lambda j: bmv[j, :],
            lambda j: j * 16 + iota16,
            _init_carry())
        gidv[...] = bi
        pltpu.async_copy(q2_hbm.at[gidv], cand, sem).wait()
        gv = gidv[...]
        carry = _init_carry()
        for r in range(K):
            base = gv[r] * G
            carry = _scan_topk(
                G // 16,
                lambda j, r=r: cand[r, pl.ds(j * 16, 16)],
                lambda j, base=base: base + j * 16 + iota16,
                carry)
        biv[...] = carry[1]
        pltpu.async_copy(train_hbm.at[biv], rows, sem).wait()
        pltpu.sync_copy(rows, p_out)


def _k2(q2, qbm, train):
    mesh = plsc.VectorSubcoreMesh(**_SC_MESH)
    kern = functools.partial(
        pl.kernel,
        mesh=mesh,
        out_type=jax.ShapeDtypeStruct((K, D), jnp.float32),
        scratch_types=[
            pltpu.VMEM((NB_V, 16), jnp.float32),
            pltpu.VMEM((16,), jnp.int32),
            pltpu.VMEM((K, G), jnp.float32),
            pltpu.VMEM((16,), jnp.int32),
            pltpu.VMEM((K, D), jnp.float32),
            pltpu.SemaphoreType.DMA,
        ],
        compiler_params=_SC_PARAMS,
    )(_k2_body)
    return kern(q2, qbm, train)


def _k3_body(pur, tr, smr, bmr):
    i = pl.program_id(0)
    u, valid = _aug(tr, i)
    ss = lax.dot_general(pur[...], u, _NT,
                         preferred_element_type=jnp.float32)
    sblk = jnp.where(valid, ss, jnp.float32(jnp.inf))
    _store_grouped(sblk, smr, K)
    mins = [jnp.min(sblk[:, g * G:(g + 1) * G], axis=1, keepdims=True)
            for g in range(GPB)]
    bmr[...] = jnp.concatenate(mins, axis=1)[None]


def _k3(pu, train):
    return pl.pallas_call(
        _k3_body,
        grid=(NGRID,),
        in_specs=[
            pl.BlockSpec((K, 2 * D), lambda i: (0, 0)),
            pl.BlockSpec((BK, D), lambda i: (i, 0)),
        ],
        out_specs=[
            pl.BlockSpec((K, GPB, G), lambda i: (0, i, 0)),
            pl.BlockSpec((1, K, GPB), lambda i: (i, 0, 0)),
        ],
        out_shape=[
            jax.ShapeDtypeStruct((K, NB, G), jnp.float32),
            jax.ShapeDtypeStruct((NGRID, K, GPB), jnp.float32),
        ],
        compiler_params=pltpu.CompilerParams(
            dimension_semantics=("arbitrary",)),
    )(pu, train)


def _k4_body(sm2_hbm, bm_hbm, sb_out,
             bmv, gids, cand, bvv, sem):
    c = lax.axis_index("c")
    s = lax.axis_index("s")
    wid = s * 2 + c

    @pl.when(wid < K)
    def _():
        pltpu.sync_copy(bm_hbm.at[:, wid, :], bmv)
        iota16 = lax.iota(jnp.int32, 16)
        _, bi, _ = _scan_topk(
            NB_V,
            lambda j: bmv[j >> 1, pl.ds((j & 1) * 16, 16)],
            lambda j: j * 16 + iota16,
            _init_carry())
        gids[...] = bi + wid * NB
        pltpu.async_copy(sm2_hbm.at[gids], cand, sem).wait()
        carry = _init_carry()
        for r in range(K):
            carry = _scan_topk(
                G // 16,
                lambda j, r=r: cand[r, pl.ds(j * 16, 16)],
                lambda j: iota16,
                carry)
        bvv[...] = carry[0]
        pltpu.sync_copy(bvv, sb_out.at[wid])


def _k4(sm2, bm):
    mesh = plsc.VectorSubcoreMesh(**_SC_MESH)
    kern = functools.partial(
        pl.kernel,
        mesh=mesh,
        out_type=jax.ShapeDtypeStruct((K, 16), jnp.float32),
        scratch_types=[
            pltpu.VMEM((NGRID, GPB), jnp.float32),
            pltpu.VMEM((16,), jnp.int32),
            pltpu.VMEM((K, G), jnp.float32),
            pltpu.VMEM((16,), jnp.float32),
            pltpu.SemaphoreType.DMA,
        ],
        compiler_params=_SC_PARAMS,
    )(_k4_body)
    return kern(sm2, bm)


def _k5_body(xr, pr, sbr, orf):
    x = xr[...]
    p = pr[...]
    diff = p - x
    d2x = jnp.sum(diff * diff, axis=1, keepdims=True)
    pd = LAMBDA * jnp.sqrt(jnp.sum(d2x, axis=0, keepdims=True) / K)
    pnorm = jnp.sum(p * p, axis=1, keepdims=True)
    sums = jnp.sum(sbr[...], axis=1, keepdims=True) + K * pnorm
    pdp = LAMBDA * jnp.sqrt(sums / K)
    nf = jnp.sum(pdp, axis=0, keepdims=True)
    lof = pd / nf * K - 1.0
    z = lof * jnp.float32(1.0 / math.sqrt(2.0))
    az = jnp.abs(z)
    t = 1.0 / (1.0 + 0.3275911 * az)
    poly = t * (0.254829592 + t * (-0.284496736 + t * (
        1.421413741 + t * (-1.453152027 + t * 1.061405429))))
    erf_abs = 1.0 - poly * jnp.exp(-az * az)
    erfz = jnp.where(z >= 0, erf_abs, -erf_abs)
    orf[...] = jnp.maximum(jnp.float32(0.0), erfz)


def _k5(xp, p, sb):
    return pl.pallas_call(
        _k5_body,
        in_specs=[
            pl.BlockSpec((1, D), lambda: (0, 0)),
            pl.BlockSpec((K, D), lambda: (0, 0)),
            pl.BlockSpec((K, 16), lambda: (0, 0)),
        ],
        out_specs=pl.BlockSpec((1, 1), lambda: (0, 0)),
        out_shape=jax.ShapeDtypeStruct((1, 1), jnp.float32),
    )(xp, p, sb)


def kernel(X, train_points):
    X = X.astype(jnp.float32)
    train_points = train_points.astype(jnp.float32)
    xu = jnp.zeros((8, 2 * D), jnp.float32)
    xu = xu.at[0, :D].set(1.0).at[0, D:].set(-2.0 * X)
    q3, qbm3 = _k1(xu, train_points)
    q2 = q3.reshape(NB, G)
    qbm2 = qbm3.reshape(NB_V, 16)
    p = _k2(q2, qbm2, train_points)
    pu = jnp.concatenate([jnp.ones((K, D), jnp.float32), -2.0 * p], axis=1)
    sm3, bm = _k3(pu, train_points)
    sb = _k4(sm3.reshape(K * NB, G), bm)
    out = _k5(X[None, :], p, sb)
    return out.reshape(())


# speedup vs baseline: 3.4870x; 3.4870x over previous
import functools
import math

import jax
import jax.numpy as jnp
from jax import lax
from jax.experimental import pallas as pl
from jax.experimental.pallas import tpu as pltpu
from jax.experimental.pallas import tpu_sc as plsc

N = 1000000
D = 64
K = 16
LAMBDA = 3.0

BK = 8192
NGRID = 123
NTOT = NGRID * BK
G = 256
GPB = BK // G
NB = NTOT // G
NB_V = NB // 16

_NN = (((1,), (0,)), ((), ()))


def _mask_pad(vals, i, nrows):
    idx = jax.lax.broadcasted_iota(jnp.int32, (nrows, BK), 1) + i * BK
    return jnp.where(idx < N, vals, jnp.float32(jnp.inf))


def _store_grouped(vals, out_ref, bmr):
    for g in range(GPB):
        out_ref[:, g, :] = vals[:, g * G:(g + 1) * G]
    mins = [jnp.min(vals[:, g * G:(g + 1) * G], axis=1, keepdims=True)
            for g in range(GPB)]
    bmr[...] = jnp.concatenate(mins, axis=1)[None]


def _k1_body(xr, ttr, qr, bmr):
    i = pl.program_id(0)
    tt = ttr[...]
    xb = xr[...]
    q = jnp.sum(tt * (tt - 2.0 * xb), axis=0, keepdims=True)
    q = _mask_pad(q, i, 1)
    _store_grouped(q, qr, bmr)


def _k1(xcol, tt):
    return pl.pallas_call(
        _k1_body,
        grid=(NGRID,),
        in_specs=[
            pl.BlockSpec((D, 1), lambda i: (0, 0)),
            pl.BlockSpec((D, BK), lambda i: (0, i)),
        ],
        out_specs=[
            pl.BlockSpec((1, GPB, G), lambda i: (i, 0, 0)),
            pl.BlockSpec((1, 1, GPB), lambda i: (i, 0, 0)),
        ],
        out_shape=[
            jax.ShapeDtypeStruct((NGRID, GPB, G), jnp.float32),
            jax.ShapeDtypeStruct((NGRID, 1, GPB), jnp.float32),
        ],
        compiler_params=pltpu.CompilerParams(
            dimension_semantics=("arbitrary",)),
    )(xcol, tt)


def _merge16(bv, bi, v, iv):
    sv, si = plsc.sort_key_val(v, iv)
    rv = lax.rev(sv, (0,))
    ri = lax.rev(si, (0,))
    take = bv <= rv
    lo = jnp.minimum(bv, rv)
    li = jnp.where(take, bi, ri)
    nbv, nbi = plsc.sort_key_val(lo, li)
    return nbv, nbi, nbv[15]


def _scan_topk(nvecs, getv, geti, carry):
    def body(j, c):
        bv, bi, th = c
        v = getv(j)
        pred = jnp.any(v < th)

        def do(_):
            return _merge16(bv, bi, v, geti(j))

        return lax.cond(pred, do, lambda _: (bv, bi, th), 0)

    return lax.fori_loop(0, nvecs, body, carry)


def _init_carry():
    inf = jnp.float32(jnp.inf)
    return (jnp.full((16,), inf, jnp.float32),
            jnp.zeros((16,), jnp.int32),
            inf)


_SC_PARAMS = pltpu.CompilerParams(
    needs_layout_passes=False, use_tc_tiling_on_sc=False)
_SC_MESH = dict(core_axis_name="c", subcore_axis_name="s")


def _k2_body(q2_hbm, qbm_hbm, idx_out,
             bmv, gidv, cand, biv, sem):
    c = lax.axis_index("c")
    s = lax.axis_index("s")
    wid = s * 2 + c

    @pl.when(wid == 0)
    def _():
        pltpu.sync_copy(qbm_hbm, bmv)
        iota16 = lax.iota(jnp.int32, 16)
        _, bi, _ = _scan_topk(
            NB_V,
            lambda j: bmv[j, :],
            lambda j: j * 16 + iota16,
            _init_carry())
        gidv[...] = bi
        pltpu.async_copy(q2_hbm.at[gidv], cand, sem).wait()
        gv = gidv[...]
        carry = _init_carry()
        for r in range(K):
            base = gv[r] * G
            carry = _scan_topk(
                G // 16,
                lambda j, r=r: cand[r, pl.ds(j * 16, 16)],
                lambda j, base=base: base + j * 16 + iota16,
                carry)
        biv[...] = carry[1]
        pltpu.sync_copy(biv, idx_out)


def _k2(q2, qbm):
    mesh = plsc.VectorSubcoreMesh(**_SC_MESH)
    kern = functools.partial(
        pl.kernel,
        mesh=mesh,
        out_type=jax.ShapeDtypeStruct((16,), jnp.int32),
        scratch_types=[
            pltpu.VMEM((NB_V, 16), jnp.float32),
            pltpu.VMEM((16,), jnp.int32),
            pltpu.VMEM((K, G), jnp.float32),
            pltpu.VMEM((16,), jnp.int32),
            pltpu.SemaphoreType.DMA,
        ],
        compiler_params=_SC_PARAMS,
    )(_k2_body)
    return kern(q2, qbm)


def _k2c_body(idxr, tt_any, pt_ref, buf, sem):
    lanes = jax.lax.broadcasted_iota(jnp.int32, (D, 128), 1)
    for r in range(K):
        col = idxr[r]
        colbase = pl.multiple_of((col // 128) * 128, 128)
        cp = pltpu.make_async_copy(
            tt_any.at[:, pl.ds(colbase, 128)], buf, sem)
        cp.start()
        cp.wait()
        sel = jnp.where(lanes == col % 128, buf[...], jnp.float32(0.0))
        pt_ref[:, r:r + 1] = jnp.sum(sel, axis=1, keepdims=True)


def _k2c(idx, tt):
    return pl.pallas_call(
        _k2c_body,
        in_specs=[
            pl.BlockSpec(memory_space=pltpu.SMEM),
            pl.BlockSpec(memory_space=pltpu.MemorySpace.HBM),
        ],
        out_specs=pl.BlockSpec((D, K), lambda: (0, 0)),
        out_shape=jax.ShapeDtypeStruct((D, K), jnp.float32),
        scratch_shapes=[pltpu.VMEM((D, 128), jnp.float32),
                        pltpu.SemaphoreType.DMA],
    )(idx, tt)


def _k3_body(pur, ttr, smr, bmr):
    i = pl.program_id(0)
    tt = ttr[...]
    u = jnp.concatenate([tt * tt, tt], axis=0)
    ss = lax.dot_general(pur[...], u, _NN,
                         preferred_element_type=jnp.float32)
    sblk = _mask_pad(ss, i, K)
    _store_grouped(sblk, smr, bmr)


def _k3(pu, tt):
    return pl.pallas_call(
        _k3_body,
        grid=(NGRID,),
        in_specs=[
            pl.BlockSpec((K, 2 * D), lambda i: (0, 0)),
            pl.BlockSpec((D, BK), lambda i: (0, i)),
        ],
        out_specs=[
            pl.BlockSpec((K, GPB, G), lambda i: (0, i, 0)),
            pl.BlockSpec((1, K, GPB), lambda i: (i, 0, 0)),
        ],
        out_shape=[
            jax.ShapeDtypeStruct((K, NB, G), jnp.float32),
            jax.ShapeDtypeStruct((NGRID, K, GPB), jnp.float32),
        ],
        compiler_params=pltpu.CompilerParams(
            dimension_semantics=("arbitrary",)),
    )(pu, tt)


def _k4_body(sm2_hbm, bm_hbm, sb_out,
             bmv, gids, cand, bvv, sem):
    c = lax.axis_index("c")
    s = lax.axis_index("s")
    wid = s * 2 + c

    @pl.when(wid < K)
    def _():
        pltpu.sync_copy(bm_hbm.at[:, wid, :], bmv)
        iota16 = lax.iota(jnp.int32, 16)
        _, bi, _ = _scan_topk(
            NB_V,
            lambda j: bmv[j >> 1, pl.ds((j & 1) * 16, 16)],
            lambda j: j * 16 + iota16,
            _init_carry())
        gids[...] = bi + wid * NB
        pltpu.async_copy(sm2_hbm.at[gids], cand, sem).wait()
        carry = _init_carry()
        for r in range(K):
            carry = _scan_topk(
                G // 16,
                lambda j, r=r: cand[r, pl.ds(j * 16, 16)],
                lambda j: iota16,
                carry)
        bvv[...] = carry[0]
        pltpu.sync_copy(bvv, sb_out.at[wid])


def _k4(sm2, bm):
    mesh = plsc.VectorSubcoreMesh(**_SC_MESH)
    kern = functools.partial(
        pl.kernel,
        mesh=mesh,
        out_type=jax.ShapeDtypeStruct((K, 16), jnp.float32),
        scratch_types=[
            pltpu.VMEM((NGRID, GPB), jnp.float32),
            pltpu.VMEM((16,), jnp.int32),
            pltpu.VMEM((K, G), jnp.float32),
            pltpu.VMEM((16,), jnp.float32),
            pltpu.SemaphoreType.DMA,
        ],
        compiler_params=_SC_PARAMS,
    )(_k4_body)
    return kern(sm2, bm)


def _k5_body(xr, ptr, sbtr, orf):
    x = xr[...]
    pt = ptr[...]
    diff = pt - x
    d2x = jnp.sum(diff * diff, axis=0, keepdims=True)
    pd = LAMBDA * jnp.sqrt(jnp.sum(d2x, axis=1, keepdims=True) / K)
    pnorm = jnp.sum(pt * pt, axis=0, keepdims=True)
    sums = jnp.sum(sbtr[...], axis=0, keepdims=True) + K * pnorm
    pdp = LAMBDA * jnp.sqrt(sums / K)
    nf = jnp.sum(pdp, axis=1, keepdims=True)
    lof = pd / nf * K - 1.0
    z = lof * jnp.float32(1.0 / math.sqrt(2.0))
    az = jnp.abs(z)
    t = 1.0 / (1.0 + 0.3275911 * az)
    poly = t * (0.254829592 + t * (-0.284496736 + t * (
        1.421413741 + t * (-1.453152027 + t * 1.061405429))))
    erf_abs = 1.0 - poly * jnp.exp(-az * az)
    erfz = jnp.where(z >= 0, erf_abs, -erf_abs)
    orf[...] = jnp.maximum(jnp.float32(0.0), erfz)


def _k5(xcol, pt, sbt):
    return pl.pallas_call(
        _k5_body,
        in_specs=[
            pl.BlockSpec((D, 1), lambda: (0, 0)),
            pl.BlockSpec((D, K), lambda: (0, 0)),
            pl.BlockSpec((16, K), lambda: (0, 0)),
        ],
        out_specs=pl.BlockSpec((1, 1), lambda: (0, 0)),
        out_shape=jax.ShapeDtypeStruct((1, 1), jnp.float32),
    )(xcol, pt, sbt)


def kernel(X, train_points):
    X = X.astype(jnp.float32)
    tt = jnp.transpose(train_points.astype(jnp.float32))
    xcol = X[:, None]
    q3, qbm3 = _k1(xcol, tt)
    q2 = q3.reshape(NB, G)
    qbm2 = qbm3.reshape(NB_V, 16)
    nidx = _k2(q2, qbm2)
    pt = _k2c(nidx, tt)
    pu = jnp.concatenate(
        [jnp.ones((K, D), jnp.float32), -2.0 * jnp.transpose(pt)], axis=1)
    sm3, bm = _k3(pu, tt)
    sb = _k4(sm3.reshape(K * NB, G), bm)
    out = _k5(xcol, pt, jnp.transpose(sb))
    return out.reshape(())
